# baseline (speedup 1.0000x reference)
import functools

import jax
import jax.numpy as jnp
from jax import lax
from jax.experimental import pallas as pl
from jax.experimental.pallas import tpu as pltpu
from jax.experimental.pallas import tpu_sc as plsc

_N = 4096
_K = 10
_INV_T = 10.0
_L = 16
_NC, _NS = 2, 16
_NW = _NC * _NS
_RPW = _N // _NW
_NCHUNK = _N // _L
_NGRP = _NCHUNK // _L


def _merge_desc(av, ai, bv, bi):
    rv = lax.rev(bv, (0,))
    ri = lax.rev(bi, (0,))
    take = rv > av
    nv = jnp.where(take, rv, av)
    ni = jnp.where(take, ri, ai)
    return plsc.sort_key_val(nv, ni, descending=True)


def _merge_unsorted(av, ai, v, i):
    sv, si = plsc.sort_key_val(v, i, descending=False)
    take = sv > av
    nv = jnp.where(take, sv, av)
    ni = jnp.where(take, si, ai)
    return plsc.sort_key_val(nv, ni, descending=True)


def _top16_of_vregs(pairs, n_acc):
    accs = []
    for a in range(n_acc):
        av = jnp.full((_L,), -2.0, jnp.float32)
        ai = jnp.zeros((_L,), jnp.int32)
        for p in range(a, len(pairs), n_acc):
            av, ai = _merge_unsorted(av, ai, *pairs[p])
        accs.append((av, ai))
    while len(accs) > 1:
        accs = [_merge_desc(*accs[i], *accs[i + 1])
                for i in range(0, len(accs), 2)]
    return accs[0]


_B = 8
_NBATCH = _RPW // _B


_PB = 4


def _sc_body(init_hbm, pred_hbm, out_hbm,
             ib0, ib1, pb0, pb1, val_blk, mbuf,
             si0, si1, sp0, sp1):
    wid = lax.axis_index("s") * _NC + lax.axis_index("c")
    base = wid * _RPW
    lanes = lax.iota(jnp.int32, _L)
    lastb = base + _RPW - _B
    lastp = base + _RPW - _PB

    pltpu.async_copy(init_hbm.at[pl.ds(base, _B)], ib0, si0)
    pltpu.async_copy(init_hbm.at[pl.ds(base + _B, _B)], ib1, si1)
    pltpu.async_copy(pred_hbm.at[pl.ds(base, _PB)], pb0, sp0)
    pltpu.async_copy(pred_hbm.at[pl.ds(base + _PB, _PB)], pb1, sp1)

    def stage1(row, r, buf, mbuf):
        rvec = jnp.full((_L,), r, jnp.int32)
        plsc.store_scatter(buf, [rvec, jnp.full((_L,), row, jnp.int32)],
                           jnp.full((_L,), -1.0, jnp.float32),
                           mask=lanes == 0)

        def grp(g, _):
            m = None
            for j in range(_L):
                e = plsc.load_gather(buf, [rvec, g * 256 + lanes * _L + j])
                m = e if m is None else jnp.maximum(m, e)
            mbuf[pl.ds(g * _L, _L)] = m
            return 0

        lax.fori_loop(0, _NGRP, grp, 0)

    def stage23(row, r, rp, buf, pbuf, mbuf):
        rvec = jnp.full((_L,), r, jnp.int32)
        grp_pairs = [(mbuf[pl.ds(g * _L, _L)], g * _L + lanes)
                     for g in range(_NGRP)]
        _, cid = _top16_of_vregs(grp_pairs, 4)
        cbase = cid * _L
        cand_pairs = []
        for j in range(_L):
            cols = cbase + j
            cand_pairs.append((plsc.load_gather(buf, [rvec, cols]), cols))
        _, fcol = _top16_of_vregs(cand_pairs, 4)
        val_blk[row - base] = \
            plsc.load_gather(pbuf, [jnp.full((_L,), rp, jnp.int32), fcol])

    def do_batch(t, ibuf, isem, mbuf):
        row0 = base + t * _B
        pltpu.make_async_copy(init_hbm.at[pl.ds(row0, _B)], ibuf, isem).wait()

        def half(h, pbuf, psem):
            r0 = row0 + h * _PB
            pltpu.make_async_copy(pred_hbm.at[pl.ds(r0, _PB)],
                                  pbuf, psem).wait()

            def inner(u, _):
                stage1(r0 + u, h * _PB + u, ibuf, mbuf)
                stage23(r0 + u, h * _PB + u, u, ibuf, pbuf, mbuf)
                return 0

            lax.fori_loop(0, _PB, inner, 0)
            nxp = jnp.minimum(r0 + _B, lastp)
            pltpu.async_copy(pred_hbm.at[pl.ds(nxp, _PB)], pbuf, psem)

        half(0, pb0, sp0)
        half(1, pb1, sp1)
        nxt = jnp.minimum(row0 + 2 * _B, lastb)
        pltpu.async_copy(init_hbm.at[pl.ds(nxt, _B)], ibuf, isem)

    def quad_body(q, _):
        do_batch(2 * q, ib0, si0, mbuf)
        do_batch(2 * q + 1, ib1, si1, mbuf)
        return 0

    lax.fori_loop(0, _NBATCH // 2, quad_body, 0)
    pltpu.make_async_copy(init_hbm.at[pl.ds(lastb, _B)], ib0, si0).wait()
    pltpu.make_async_copy(init_hbm.at[pl.ds(lastb, _B)], ib1, si1).wait()
    pltpu.make_async_copy(pred_hbm.at[pl.ds(lastp, _PB)], pb0, sp0).wait()
    pltpu.make_async_copy(pred_hbm.at[pl.ds(lastp, _PB)], pb1, sp1).wait()

    pltpu.sync_copy(val_blk, out_hbm.at[pl.ds(base, _RPW)])


@functools.cache
def _sc_topk_gather():
    return functools.partial(
        pl.kernel,
        out_type=jax.ShapeDtypeStruct((_N, _L), jnp.float32),
        mesh=plsc.VectorSubcoreMesh(core_axis_name="c", subcore_axis_name="s",
                                    num_cores=_NC, num_subcores=_NS),
        scratch_types=[
            pltpu.VMEM((_B, _N), jnp.float32),
            pltpu.VMEM((_B, _N), jnp.float32),
            pltpu.VMEM((_PB, _N), jnp.float32),
            pltpu.VMEM((_PB, _N), jnp.float32),
            pltpu.VMEM((_RPW, _L), jnp.float32),
            pltpu.VMEM((_NCHUNK,), jnp.float32),
            pltpu.SemaphoreType.DMA,
            pltpu.SemaphoreType.DMA,
            pltpu.SemaphoreType.DMA,
            pltpu.SemaphoreType.DMA,
        ],
        compiler_params=pltpu.CompilerParams(needs_layout_passes=False),
    )(_sc_body)


_R = 128
_G = _N // _R


def _tc_dense_body(aff, xt, xp, z, zp, lm_ref, ls_ref, den_ref, acc):
    g = pl.program_id(0)

    @pl.when(g == 0)
    def _():
        acc[0] = 0.0
        acc[1] = 0.0

    rows = g * _R + lax.broadcasted_iota(jnp.int32, (_R, _N), 0)
    cols = lax.broadcasted_iota(jnp.int32, (_R, _N), 1)
    sim = jnp.exp(aff[...] * _INV_T)
    sim = jnp.where(cols == rows, 0.0, sim)
    den_ref[...] = jnp.sum(sim, axis=1, keepdims=True)

    dx = xp[...] - xt[...]
    dz = zp[...] - z[...]
    acc[0] = acc[0] + jnp.sum(dx * dx)
    acc[1] = acc[1] + jnp.sum(dz * dz)

    @pl.when(g == _G - 1)
    def _():
        lm_ref[0, 0] = acc[0] / (_N * 1024.0)
        ls_ref[0, 0] = acc[1] / (_N * 256.0)


_tc_dense = pl.pallas_call(
    _tc_dense_body,
    grid=(_G,),
    in_specs=[
        pl.BlockSpec((_R, _N), lambda g: (g, 0)),
        pl.BlockSpec((_R, 1024), lambda g: (g, 0)),
        pl.BlockSpec((_R, 1024), lambda g: (g, 0)),
        pl.BlockSpec((_R, 256), lambda g: (g, 0)),
        pl.BlockSpec((_R, 256), lambda g: (g, 0)),
    ],
    out_specs=[
        pl.BlockSpec(memory_space=pltpu.SMEM),
        pl.BlockSpec(memory_space=pltpu.SMEM),
        pl.BlockSpec((_R, 1), lambda g: (g, 0)),
    ],
    out_shape=[
        jax.ShapeDtypeStruct((1, 1), jnp.float32),
        jax.ShapeDtypeStruct((1, 1), jnp.float32),
        jax.ShapeDtypeStruct((_N, 1), jnp.float32),
    ],
    scratch_shapes=[pltpu.SMEM((2,), jnp.float32)],
)


def _tc_reg_body(pp, den, lr_ref):
    pos = jnp.exp(pp[...] * _INV_T)
    terms = -jnp.log(pos / den[...] + 1e-8)
    kmask = lax.broadcasted_iota(jnp.int32, (_N, _L), 1) < _K
    lr_ref[0, 0] = jnp.sum(jnp.where(kmask, terms, 0.0)) / _N


_tc_reg = pl.pallas_call(
    _tc_reg_body,
    in_specs=[
        pl.BlockSpec((_N, _L), lambda: (0, 0)),
        pl.BlockSpec((_N, 1), lambda: (0, 0)),
    ],
    out_specs=pl.BlockSpec(memory_space=pltpu.SMEM),
    out_shape=jax.ShapeDtypeStruct((1, 1), jnp.float32),
)


def kernel(affinity_mat, aff_init, x_true, x_predict, z, z_pred, eta):
    pos_pred = _sc_topk_gather()(aff_init, affinity_mat)
    lm, ls, den = _tc_dense(affinity_mat, x_true, x_predict, z, z_pred)
    lr = _tc_reg(pos_pred, den)
    lm = lm[0, 0]
    ls = ls[0, 0]
    lr = lr[0, 0]
    ene = jnp.exp(-eta)
    loss = jnp.sum(jnp.stack([lm, ls, lr]) * ene + eta)
    return (loss, lm, ls, lr, ene)


# speedup vs baseline: 1.3612x; 1.3612x over previous
import functools

import jax
import jax.numpy as jnp
from jax import lax
from jax.experimental import pallas as pl
from jax.experimental.pallas import tpu as pltpu
from jax.experimental.pallas import tpu_sc as plsc

_N = 4096
_K = 10
_INV_T = 10.0
_L = 16
_NC, _NS = 2, 16
_NW = _NC * _NS
_RPW = _N // _NW
_NCHUNK = _N // _L
_NGRP = _NCHUNK // _L


def _merge_desc(av, ai, bv, bi):
    rv = lax.rev(bv, (0,))
    ri = lax.rev(bi, (0,))
    take = rv > av
    nv = jnp.where(take, rv, av)
    ni = jnp.where(take, ri, ai)
    return plsc.sort_key_val(nv, ni, descending=True)


def _merge_unsorted(av, ai, v, i):
    sv, si = plsc.sort_key_val(v, i, descending=False)
    take = sv > av
    nv = jnp.where(take, sv, av)
    ni = jnp.where(take, si, ai)
    return plsc.sort_key_val(nv, ni, descending=True)


def _top16_of_vregs(pairs, n_acc):
    accs = []
    for a in range(n_acc):
        av = jnp.full((_L,), -2.0, jnp.float32)
        ai = jnp.zeros((_L,), jnp.int32)
        for p in range(a, len(pairs), n_acc):
            av, ai = _merge_unsorted(av, ai, *pairs[p])
        accs.append((av, ai))
    while len(accs) > 1:
        accs = [_merge_desc(*accs[i], *accs[i + 1])
                for i in range(0, len(accs), 2)]
    return accs[0]


_B = 8
_NBATCH = _RPW // _B


_PB = 4


def _sc_body(init_hbm, pred_hbm, out_hbm,
             ib0, ib1, pb0, pb1, val_blk, mbuf,
             si0, si1, sp0, sp1):
    wid = lax.axis_index("s") * _NC + lax.axis_index("c")
    base = wid * _RPW
    lanes = lax.iota(jnp.int32, _L)
    lastb = base + _RPW - _B
    lastp = base + _RPW - _PB

    pltpu.async_copy(init_hbm.at[pl.ds(base, _B)], ib0, si0)
    pltpu.async_copy(init_hbm.at[pl.ds(base + _B, _B)], ib1, si1)
    pltpu.async_copy(pred_hbm.at[pl.ds(base, _PB)], pb0, sp0)
    pltpu.async_copy(pred_hbm.at[pl.ds(base + _PB, _PB)], pb1, sp1)

    def stage1(row, r, buf, mbuf):
        rvec = jnp.full((_L,), r, jnp.int32)
        plsc.store_scatter(buf, [rvec, jnp.full((_L,), row, jnp.int32)],
                           jnp.full((_L,), -1.0, jnp.float32),
                           mask=lanes == 0)

        def grp(g, _):
            m = None
            for k in range(_L):
                e = buf[r, pl.ds(g * 256 + k * _L, _L)]
                m = e if m is None else jnp.maximum(m, e)
            mbuf[pl.ds(g * _L, _L)] = m
            return 0

        lax.fori_loop(0, _NGRP, grp, 0)

    def stage23(row, r, rp, buf, pbuf, mbuf):
        rvec = jnp.full((_L,), r, jnp.int32)
        grp_pairs = [(mbuf[pl.ds(g * _L, _L)], g * _L + lanes)
                     for g in range(_NGRP)]
        _, cid = _top16_of_vregs(grp_pairs, 4)
        cbase = (cid // _L) * 256 + (cid % _L)
        cand_pairs = []
        for j in range(_L):
            cols = cbase + j * _L
            cand_pairs.append((plsc.load_gather(buf, [rvec, cols]), cols))
        _, fcol = _top16_of_vregs(cand_pairs, 4)
        val_blk[row - base] = \
            plsc.load_gather(pbuf, [jnp.full((_L,), rp, jnp.int32), fcol])

    def do_batch(t, ibuf, isem, mbuf):
        row0 = base + t * _B
        pltpu.make_async_copy(init_hbm.at[pl.ds(row0, _B)], ibuf, isem).wait()

        def half(h, pbuf, psem):
            r0 = row0 + h * _PB
            pltpu.make_async_copy(pred_hbm.at[pl.ds(r0, _PB)],
                                  pbuf, psem).wait()

            def inner(u, _):
                stage1(r0 + u, h * _PB + u, ibuf, mbuf)
                stage23(r0 + u, h * _PB + u, u, ibuf, pbuf, mbuf)
                return 0

            lax.fori_loop(0, _PB, inner, 0)
            nxp = jnp.minimum(r0 + _B, lastp)
            pltpu.async_copy(pred_hbm.at[pl.ds(nxp, _PB)], pbuf, psem)

        half(0, pb0, sp0)
        half(1, pb1, sp1)
        nxt = jnp.minimum(row0 + 2 * _B, lastb)
        pltpu.async_copy(init_hbm.at[pl.ds(nxt, _B)], ibuf, isem)

    def quad_body(q, _):
        do_batch(2 * q, ib0, si0, mbuf)
        do_batch(2 * q + 1, ib1, si1, mbuf)
        return 0

    lax.fori_loop(0, _NBATCH // 2, quad_body, 0)
    pltpu.make_async_copy(init_hbm.at[pl.ds(lastb, _B)], ib0, si0).wait()
    pltpu.make_async_copy(init_hbm.at[pl.ds(lastb, _B)], ib1, si1).wait()
    pltpu.make_async_copy(pred_hbm.at[pl.ds(lastp, _PB)], pb0, sp0).wait()
    pltpu.make_async_copy(pred_hbm.at[pl.ds(lastp, _PB)], pb1, sp1).wait()

    pltpu.sync_copy(val_blk, out_hbm.at[pl.ds(base, _RPW)])


@functools.cache
def _sc_topk_gather():
    return functools.partial(
        pl.kernel,
        out_type=jax.ShapeDtypeStruct((_N, _L), jnp.float32),
        mesh=plsc.VectorSubcoreMesh(core_axis_name="c", subcore_axis_name="s",
                                    num_cores=_NC, num_subcores=_NS),
        scratch_types=[
            pltpu.VMEM((_B, _N), jnp.float32),
            pltpu.VMEM((_B, _N), jnp.float32),
            pltpu.VMEM((_PB, _N), jnp.float32),
            pltpu.VMEM((_PB, _N), jnp.float32),
            pltpu.VMEM((_RPW, _L), jnp.float32),
            pltpu.VMEM((_NCHUNK,), jnp.float32),
            pltpu.SemaphoreType.DMA,
            pltpu.SemaphoreType.DMA,
            pltpu.SemaphoreType.DMA,
            pltpu.SemaphoreType.DMA,
        ],
        compiler_params=pltpu.CompilerParams(needs_layout_passes=False),
    )(_sc_body)


_R = 128
_G = _N // _R


def _tc_dense_body(aff, xt, xp, z, zp, lm_ref, ls_ref, den_ref, acc):
    g = pl.program_id(0)

    @pl.when(g == 0)
    def _():
        acc[0] = 0.0
        acc[1] = 0.0

    rows = g * _R + lax.broadcasted_iota(jnp.int32, (_R, _N), 0)
    cols = lax.broadcasted_iota(jnp.int32, (_R, _N), 1)
    sim = jnp.exp(aff[...] * _INV_T)
    sim = jnp.where(cols == rows, 0.0, sim)
    den_ref[...] = jnp.sum(sim, axis=1, keepdims=True)

    dx = xp[...] - xt[...]
    dz = zp[...] - z[...]
    acc[0] = acc[0] + jnp.sum(dx * dx)
    acc[1] = acc[1] + jnp.sum(dz * dz)

    @pl.when(g == _G - 1)
    def _():
        lm_ref[0, 0] = acc[0] / (_N * 1024.0)
        ls_ref[0, 0] = acc[1] / (_N * 256.0)


_tc_dense = pl.pallas_call(
    _tc_dense_body,
    grid=(_G,),
    in_specs=[
        pl.BlockSpec((_R, _N), lambda g: (g, 0)),
        pl.BlockSpec((_R, 1024), lambda g: (g, 0)),
        pl.BlockSpec((_R, 1024), lambda g: (g, 0)),
        pl.BlockSpec((_R, 256), lambda g: (g, 0)),
        pl.BlockSpec((_R, 256), lambda g: (g, 0)),
    ],
    out_specs=[
        pl.BlockSpec(memory_space=pltpu.SMEM),
        pl.BlockSpec(memory_space=pltpu.SMEM),
        pl.BlockSpec((_R, 1), lambda g: (g, 0)),
    ],
    out_shape=[
        jax.ShapeDtypeStruct((1, 1), jnp.float32),
        jax.ShapeDtypeStruct((1, 1), jnp.float32),
        jax.ShapeDtypeStruct((_N, 1), jnp.float32),
    ],
    scratch_shapes=[pltpu.SMEM((2,), jnp.float32)],
)


def _tc_reg_body(pp, den, lr_ref):
    pos = jnp.exp(pp[...] * _INV_T)
    terms = -jnp.log(pos / den[...] + 1e-8)
    kmask = lax.broadcasted_iota(jnp.int32, (_N, _L), 1) < _K
    lr_ref[0, 0] = jnp.sum(jnp.where(kmask, terms, 0.0)) / _N


_tc_reg = pl.pallas_call(
    _tc_reg_body,
    in_specs=[
        pl.BlockSpec((_N, _L), lambda: (0, 0)),
        pl.BlockSpec((_N, 1), lambda: (0, 0)),
    ],
    out_specs=pl.BlockSpec(memory_space=pltpu.SMEM),
    out_shape=jax.ShapeDtypeStruct((1, 1), jnp.float32),
)


def kernel(affinity_mat, aff_init, x_true, x_predict, z, z_pred, eta):
    pos_pred = _sc_topk_gather()(aff_init, affinity_mat)
    lm, ls, den = _tc_dense(affinity_mat, x_true, x_predict, z, z_pred)
    lr = _tc_reg(pos_pred, den)
    lm = lm[0, 0]
    ls = ls[0, 0]
    lr = lr[0, 0]
    ene = jnp.exp(-eta)
    loss = jnp.sum(jnp.stack([lm, ls, lr]) * ene + eta)
    return (loss, lm, ls, lr, ene)


# speedup vs baseline: 1.5157x; 1.1135x over previous
import functools

import jax
import jax.numpy as jnp
from jax import lax
from jax.experimental import pallas as pl
from jax.experimental.pallas import tpu as pltpu
from jax.experimental.pallas import tpu_sc as plsc

_N = 4096
_K = 10
_INV_T = 10.0
_L = 16
_NC, _NS = 2, 16
_NW = _NC * _NS
_RPW = _N // _NW
_NCHUNK = _N // _L
_NGRP = _NCHUNK // _L


def _merge_desc(av, ai, bv, bi):
    rv = lax.rev(bv, (0,))
    ri = lax.rev(bi, (0,))
    take = rv > av
    nv = jnp.where(take, rv, av)
    ni = jnp.where(take, ri, ai)
    return plsc.sort_key_val(nv, ni, descending=True)


def _merge_unsorted(av, ai, v, i):
    sv, si = plsc.sort_key_val(v, i, descending=False)
    take = sv > av
    nv = jnp.where(take, sv, av)
    ni = jnp.where(take, si, ai)
    return plsc.sort_key_val(nv, ni, descending=True)


def _top16_of_vregs(pairs, n_acc):
    del n_acc
    accs = [plsc.sort_key_val(v, i, descending=True) for v, i in pairs]
    while len(accs) > 1:
        odd = accs[-1] if len(accs) % 2 else None
        accs = [_merge_desc(*accs[i], *accs[i + 1])
                for i in range(0, len(accs) - 1, 2)]
        if odd is not None:
            accs.append(odd)
    return accs[0]


_B = 8
_NBATCH = _RPW // _B


_PB = 4


def _sc_body(init_hbm, pred_hbm, out_hbm,
             ib0, ib1, pb0, pb1, val_blk, mbuf,
             si0, si1, sp0, sp1):
    wid = lax.axis_index("s") * _NC + lax.axis_index("c")
    base = wid * _RPW
    lanes = lax.iota(jnp.int32, _L)
    lastb = base + _RPW - _B
    lastp = base + _RPW - _PB

    pltpu.async_copy(init_hbm.at[pl.ds(base, _B)], ib0, si0)
    pltpu.async_copy(init_hbm.at[pl.ds(base + _B, _B)], ib1, si1)
    pltpu.async_copy(pred_hbm.at[pl.ds(base, _PB)], pb0, sp0)
    pltpu.async_copy(pred_hbm.at[pl.ds(base + _PB, _PB)], pb1, sp1)

    def stage1(row, r, buf, mbuf):
        rvec = jnp.full((_L,), r, jnp.int32)
        plsc.store_scatter(buf, [rvec, jnp.full((_L,), row, jnp.int32)],
                           jnp.full((_L,), -1.0, jnp.float32),
                           mask=lanes == 0)

        def grp(g, _):
            m = None
            for k in range(_L):
                e = buf[r, pl.ds(g * 256 + k * _L, _L)]
                m = e if m is None else jnp.maximum(m, e)
            mbuf[pl.ds(g * _L, _L)] = m
            return 0

        lax.fori_loop(0, _NGRP, grp, 0)

    def stage23(row, r, rp, buf, pbuf, mbuf):
        rvec = jnp.full((_L,), r, jnp.int32)
        grp_pairs = [(mbuf[pl.ds(g * _L, _L)], g * _L + lanes)
                     for g in range(_NGRP)]
        _, cid = _top16_of_vregs(grp_pairs, 4)
        cbase = (cid // _L) * 256 + (cid % _L)
        cand_pairs = []
        for j in range(_L):
            cols = cbase + j * _L
            cand_pairs.append((plsc.load_gather(buf, [rvec, cols]), cols))
        _, fcol = _top16_of_vregs(cand_pairs, 4)
        val_blk[row - base] = \
            plsc.load_gather(pbuf, [jnp.full((_L,), rp, jnp.int32), fcol])

    def do_batch(t, ibuf, isem, mbuf):
        row0 = base + t * _B
        pltpu.make_async_copy(init_hbm.at[pl.ds(row0, _B)], ibuf, isem).wait()

        def half(h, pbuf, psem):
            r0 = row0 + h * _PB
            pltpu.make_async_copy(pred_hbm.at[pl.ds(r0, _PB)],
                                  pbuf, psem).wait()

            def inner(u, _):
                stage1(r0 + u, h * _PB + u, ibuf, mbuf)
                stage23(r0 + u, h * _PB + u, u, ibuf, pbuf, mbuf)
                return 0

            lax.fori_loop(0, _PB, inner, 0)
            nxp = jnp.minimum(r0 + _B, lastp)
            pltpu.async_copy(pred_hbm.at[pl.ds(nxp, _PB)], pbuf, psem)

        half(0, pb0, sp0)
        half(1, pb1, sp1)
        nxt = jnp.minimum(row0 + 2 * _B, lastb)
        pltpu.async_copy(init_hbm.at[pl.ds(nxt, _B)], ibuf, isem)

    def quad_body(q, _):
        do_batch(2 * q, ib0, si0, mbuf)
        do_batch(2 * q + 1, ib1, si1, mbuf)
        return 0

    lax.fori_loop(0, _NBATCH // 2, quad_body, 0)
    pltpu.make_async_copy(init_hbm.at[pl.ds(lastb, _B)], ib0, si0).wait()
    pltpu.make_async_copy(init_hbm.at[pl.ds(lastb, _B)], ib1, si1).wait()
    pltpu.make_async_copy(pred_hbm.at[pl.ds(lastp, _PB)], pb0, sp0).wait()
    pltpu.make_async_copy(pred_hbm.at[pl.ds(lastp, _PB)], pb1, sp1).wait()

    pltpu.sync_copy(val_blk, out_hbm.at[pl.ds(base, _RPW)])


@functools.cache
def _sc_topk_gather():
    return functools.partial(
        pl.kernel,
        out_type=jax.ShapeDtypeStruct((_N, _L), jnp.float32),
        mesh=plsc.VectorSubcoreMesh(core_axis_name="c", subcore_axis_name="s",
                                    num_cores=_NC, num_subcores=_NS),
        scratch_types=[
            pltpu.VMEM((_B, _N), jnp.float32),
            pltpu.VMEM((_B, _N), jnp.float32),
            pltpu.VMEM((_PB, _N), jnp.float32),
            pltpu.VMEM((_PB, _N), jnp.float32),
            pltpu.VMEM((_RPW, _L), jnp.float32),
            pltpu.VMEM((_NCHUNK,), jnp.float32),
            pltpu.SemaphoreType.DMA,
            pltpu.SemaphoreType.DMA,
            pltpu.SemaphoreType.DMA,
            pltpu.SemaphoreType.DMA,
        ],
        compiler_params=pltpu.CompilerParams(needs_layout_passes=False),
    )(_sc_body)


_R = 128
_G = _N // _R


def _tc_dense_body(aff, xt, xp, z, zp, lm_ref, ls_ref, den_ref, acc):
    g = pl.program_id(0)

    @pl.when(g == 0)
    def _():
        acc[0] = 0.0
        acc[1] = 0.0

    rows = g * _R + lax.broadcasted_iota(jnp.int32, (_R, _N), 0)
    cols = lax.broadcasted_iota(jnp.int32, (_R, _N), 1)
    sim = jnp.exp(aff[...] * _INV_T)
    sim = jnp.where(cols == rows, 0.0, sim)
    den_ref[...] = jnp.sum(sim, axis=1, keepdims=True)

    dx = xp[...] - xt[...]
    dz = zp[...] - z[...]
    acc[0] = acc[0] + jnp.sum(dx * dx)
    acc[1] = acc[1] + jnp.sum(dz * dz)

    @pl.when(g == _G - 1)
    def _():
        lm_ref[0, 0] = acc[0] / (_N * 1024.0)
        ls_ref[0, 0] = acc[1] / (_N * 256.0)


_tc_dense = pl.pallas_call(
    _tc_dense_body,
    grid=(_G,),
    in_specs=[
        pl.BlockSpec((_R, _N), lambda g: (g, 0)),
        pl.BlockSpec((_R, 1024), lambda g: (g, 0)),
        pl.BlockSpec((_R, 1024), lambda g: (g, 0)),
        pl.BlockSpec((_R, 256), lambda g: (g, 0)),
        pl.BlockSpec((_R, 256), lambda g: (g, 0)),
    ],
    out_specs=[
        pl.BlockSpec(memory_space=pltpu.SMEM),
        pl.BlockSpec(memory_space=pltpu.SMEM),
        pl.BlockSpec((_R, 1), lambda g: (g, 0)),
    ],
    out_shape=[
        jax.ShapeDtypeStruct((1, 1), jnp.float32),
        jax.ShapeDtypeStruct((1, 1), jnp.float32),
        jax.ShapeDtypeStruct((_N, 1), jnp.float32),
    ],
    scratch_shapes=[pltpu.SMEM((2,), jnp.float32)],
)


def _tc_reg_body(pp, den, lr_ref):
    pos = jnp.exp(pp[...] * _INV_T)
    terms = -jnp.log(pos / den[...] + 1e-8)
    kmask = lax.broadcasted_iota(jnp.int32, (_N, _L), 1) < _K
    lr_ref[0, 0] = jnp.sum(jnp.where(kmask, terms, 0.0)) / _N


_tc_reg = pl.pallas_call(
    _tc_reg_body,
    in_specs=[
        pl.BlockSpec((_N, _L), lambda: (0, 0)),
        pl.BlockSpec((_N, 1), lambda: (0, 0)),
    ],
    out_specs=pl.BlockSpec(memory_space=pltpu.SMEM),
    out_shape=jax.ShapeDtypeStruct((1, 1), jnp.float32),
)


def kernel(affinity_mat, aff_init, x_true, x_predict, z, z_pred, eta):
    pos_pred = _sc_topk_gather()(aff_init, affinity_mat)
    lm, ls, den = _tc_dense(affinity_mat, x_true, x_predict, z, z_pred)
    lr = _tc_reg(pos_pred, den)
    lm = lm[0, 0]
    ls = ls[0, 0]
    lr = lr[0, 0]
    ene = jnp.exp(-eta)
    loss = jnp.sum(jnp.stack([lm, ls, lr]) * ene + eta)
    return (loss, lm, ls, lr, ene)
